# baseline (speedup 1.0000x reference)
import functools

import jax
import jax.numpy as jnp
from jax.experimental import pallas as pl

_B, _S, _TL, _SL, _D, _T = 8, 32, 512, 128, 64, 32
_N = _S * _SL
_H = _N // 2


def _dot_t(a, b):
    return jax.lax.dot_general(
        a, b, (((1,), (1,)), ((), ())), preferred_element_type=jnp.float32)


def _scorer_kernel(test_ref, sup_ref, tgt_ref, out_ref, proto_ref):
    tl, d, t, n, h = _TL, _D, _T, _N, _H
    half = tl // 2

    tm2 = jnp.mean(test_ref[0], axis=0)
    tm_e = tm2[:, :d]
    tm_o = tm2[:, d:]

    supf = sup_ref[0].reshape(h, 2 * d)
    sup_e = supf[:, :d]
    sup_o = supf[:, d:]

    tgtf = tgt_ref[0].reshape(h, 2 * t)
    tgt_e = tgtf[:, :t]
    tgt_o = tgtf[:, t:]

    tag_iota = jax.lax.broadcasted_iota(jnp.int32, (h, t), 1).astype(jnp.float32)
    lab_e = jnp.sum(tgt_e * tag_iota, axis=1, keepdims=True).reshape(1, h)
    lab_o = jnp.sum(tgt_o * tag_iota, axis=1, keepdims=True).reshape(1, h)

    lane = jax.lax.broadcasted_iota(jnp.int32, (1, h), 1)
    key_e = (2 * lane) * 64 + lab_e.astype(jnp.int32)
    key_o = (2 * lane + 1) * 64 + lab_o.astype(jnp.int32)

    s2_e = jnp.sum(sup_e * sup_e, axis=1, keepdims=True).reshape(1, h)
    s2_o = jnp.sum(sup_o * sup_o, axis=1, keepdims=True).reshape(1, h)
    t2_e = jnp.sum(tm_e * tm_e, axis=1, keepdims=True)
    t2_o = jnp.sum(tm_o * tm_o, axis=1, keepdims=True)

    d2_ee = jnp.maximum(t2_e + s2_e - 2.0 * _dot_t(tm_e, sup_e), 0.0)
    d2_eo = jnp.maximum(t2_e + s2_o - 2.0 * _dot_t(tm_e, sup_o), 0.0)
    d2_oe = jnp.maximum(t2_o + s2_e - 2.0 * _dot_t(tm_o, sup_e), 0.0)
    d2_oo = jnp.maximum(t2_o + s2_o - 2.0 * _dot_t(tm_o, sup_o), 0.0)

    big = n * 64

    def winner(d2a, d2b, ka, kb):
        mv = jnp.minimum(jnp.min(d2a, axis=1, keepdims=True),
                         jnp.min(d2b, axis=1, keepdims=True))
        wa = jnp.min(jnp.where(d2a == mv, jnp.broadcast_to(ka, d2a.shape), big),
                     axis=1, keepdims=True)
        wb = jnp.min(jnp.where(d2b == mv, jnp.broadcast_to(kb, d2b.shape), big),
                     axis=1, keepdims=True)
        return jax.lax.rem(jnp.minimum(wa, wb), 64)

    wl_e = winner(d2_ee, d2_eo, key_e, key_o)
    wl_o = winner(d2_oe, d2_oo, key_e, key_o)

    out_iota = jax.lax.broadcasted_iota(jnp.int32, (half, t), 1)
    sim_e = (out_iota == wl_e).astype(jnp.float32)
    sim_o = (out_iota == wl_o).astype(jnp.float32)

    psum = (jax.lax.dot_general(tgt_e, sup_e, (((0,), (0,)), ((), ())),
                                preferred_element_type=jnp.float32)
            + jax.lax.dot_general(tgt_o, sup_o, (((0,), (0,)), ((), ())),
                                  preferred_element_type=jnp.float32))
    cnt = (jnp.sum(tgt_e, axis=0, keepdims=True)
           + jnp.sum(tgt_o, axis=0, keepdims=True))
    proto = psum / (cnt.reshape(t, 1) + 0.0001)

    sim1_e = _dot_t(tm_e, proto)
    sim1_o = _dot_t(tm_o, proto)

    out_ref[0] = jnp.concatenate(
        [sim_e + 0.5 * sim1_e, sim_o + 0.5 * sim1_o], axis=1)
    proto_ref[0] = proto


@functools.partial(jax.jit, static_argnames=())
def kernel(test_reps, support_reps, test_output_mask, support_output_mask, support_targets):
    del test_output_mask, support_output_mask
    b, s, tl, d = test_reps.shape
    sl = support_reps.shape[2]
    t = support_targets.shape[3]

    test_p = test_reps.reshape(b, s, tl // 2, 2 * d)
    sup_p = support_reps.reshape(b, s, sl // 2, 2 * d)
    tgt_p = support_targets.reshape(b, s, sl // 2, 2 * t)

    out, proto = pl.pallas_call(
        _scorer_kernel,
        grid=(b,),
        in_specs=[
            pl.BlockSpec((1, s, tl // 2, 2 * d), lambda i: (i, 0, 0, 0)),
            pl.BlockSpec((1, s, sl // 2, 2 * d), lambda i: (i, 0, 0, 0)),
            pl.BlockSpec((1, s, sl // 2, 2 * t), lambda i: (i, 0, 0, 0)),
        ],
        out_specs=[
            pl.BlockSpec((1, tl // 2, 2 * t), lambda i: (i, 0, 0)),
            pl.BlockSpec((1, t, d), lambda i: (i, 0, 0)),
        ],
        out_shape=[
            jax.ShapeDtypeStruct((b, tl // 2, 2 * t), jnp.float32),
            jax.ShapeDtypeStruct((b, t, d), jnp.float32),
        ],
    )(test_p, sup_p, tgt_p)
    return (out.reshape(b, tl, t), proto)


# speedup vs baseline: 1.5788x; 1.5788x over previous
import functools

import jax
import jax.numpy as jnp
from jax.experimental import pallas as pl


def _dot_t(a, b):
    return jax.lax.dot_general(
        a, b, (((1,), (1,)), ((), ())), preferred_element_type=jnp.float32)


def _scorer_kernel(test_ref, sup_ref, tgt_ref, out_ref, proto_ref):
    s, tl, d = test_ref.shape[1], test_ref.shape[2], test_ref.shape[3]
    sl = sup_ref.shape[2]
    t = tgt_ref.shape[3]
    n = s * sl

    tm = jnp.mean(test_ref[0], axis=0)

    sup = sup_ref[0].reshape(n, d)
    tgt = tgt_ref[0].reshape(n, t)

    t2 = jnp.sum(tm * tm, axis=1, keepdims=True)
    s2_row = jnp.sum(sup * sup, axis=1, keepdims=True).reshape(1, n)

    tagvec = jax.lax.broadcasted_iota(jnp.int32, (1, t), 1).astype(jnp.float32)
    labels_row = _dot_t(tagvec, tgt)

    lane = jax.lax.broadcasted_iota(jnp.int32, (1, n), 1)
    key_row = lane * 64 + labels_row.astype(jnp.int32)

    dot2 = _dot_t(-2.0 * tm, sup)
    d2 = jnp.maximum((t2 + s2_row) + dot2, 0.0)

    minval = jnp.min(d2, axis=1, keepdims=True)
    win = jnp.min(
        jnp.where(d2 == minval, jnp.broadcast_to(key_row, d2.shape), n * 64),
        axis=1, keepdims=True)
    win_label = jax.lax.rem(win, 64)

    out_iota = jax.lax.broadcasted_iota(jnp.int32, (tl, t), 1)
    sim = (out_iota == win_label).astype(jnp.float32)

    sup_aug = jnp.concatenate(
        [sup, jnp.ones((n, 1), dtype=jnp.float32)], axis=1)
    psum_aug = jax.lax.dot_general(
        tgt, sup_aug, (((0,), (0,)), ((), ())),
        preferred_element_type=jnp.float32)
    proto = psum_aug[:, :d] / (psum_aug[:, d:] + 0.0001)

    sim1 = _dot_t(tm, proto)

    out_ref[0] = sim + 0.5 * sim1
    proto_ref[0] = proto


@functools.partial(jax.jit, static_argnames=())
def kernel(test_reps, support_reps, test_output_mask, support_output_mask, support_targets):
    del test_output_mask, support_output_mask
    b, s, tl, d = test_reps.shape
    sl = support_reps.shape[2]
    t = support_targets.shape[3]

    out, proto = pl.pallas_call(
        _scorer_kernel,
        grid=(b,),
        in_specs=[
            pl.BlockSpec((1, s, tl, d), lambda i: (i, 0, 0, 0)),
            pl.BlockSpec((1, s, sl, d), lambda i: (i, 0, 0, 0)),
            pl.BlockSpec((1, s, sl, t), lambda i: (i, 0, 0, 0)),
        ],
        out_specs=[
            pl.BlockSpec((1, tl, t), lambda i: (i, 0, 0)),
            pl.BlockSpec((1, t, d), lambda i: (i, 0, 0)),
        ],
        out_shape=[
            jax.ShapeDtypeStruct((b, tl, t), jnp.float32),
            jax.ShapeDtypeStruct((b, t, d), jnp.float32),
        ],
    )(test_reps, support_reps, support_targets)
    return (out, proto)
